# baseline (speedup 1.0000x reference)
import jax
import jax.numpy as jnp
from jax.experimental import pallas as pl

_CODE_DIMS = 13
_NB = 4


def _vq_kernel(x_ref, win_ref, bin_ref, wout_ref, bout_ref, out_ref, idx_ref):
    rows, dim = x_ref.shape
    t = idx_ref.shape[1]
    nb = rows // t
    x = x_ref[...]
    z = jnp.dot(x, win_ref[...], preferred_element_type=jnp.float32)
    z = z + bin_ref[...][None, :]
    bits = (z > 0).astype(jnp.float32)
    q = bits * 2.0 - 1.0
    out = jnp.dot(q, wout_ref[...], preferred_element_type=jnp.float32)
    out_ref[...] = out + bout_ref[...][None, :]
    col = jax.lax.broadcasted_iota(jnp.int32, (1, _CODE_DIMS), 1)
    w_idx = jnp.exp2((_CODE_DIMS - 1 - col).astype(jnp.float32))
    i = pl.program_id(0)
    for r in range(nb):
        bits_r = bits[r * t:(r + 1) * t, :]
        idx_f = jax.lax.dot_general(
            w_idx, bits_r, (((1,), (1,)), ((), ())),
            preferred_element_type=jnp.float32,
        )
        idx_ref[pl.ds(i * nb + r, 1), :] = idx_f.astype(jnp.int32)


def kernel(x, W_in, b_in, W_out, b_out):
    B, T, DIM = x.shape
    rows = _NB * T
    out, idx = pl.pallas_call(
        _vq_kernel,
        grid=(B // _NB,),
        in_specs=[
            pl.BlockSpec((rows, DIM), lambda i: (i, 0)),
            pl.BlockSpec((DIM, _CODE_DIMS), lambda i: (0, 0)),
            pl.BlockSpec((_CODE_DIMS,), lambda i: (0,)),
            pl.BlockSpec((_CODE_DIMS, DIM), lambda i: (0, 0)),
            pl.BlockSpec((DIM,), lambda i: (0,)),
        ],
        out_specs=[
            pl.BlockSpec((rows, DIM), lambda i: (i, 0)),
            pl.BlockSpec((B, T), lambda i: (0, 0)),
        ],
        out_shape=[
            jax.ShapeDtypeStruct((B * T, DIM), jnp.float32),
            jax.ShapeDtypeStruct((B, T), jnp.int32),
        ],
    )(x.reshape(B * T, DIM), W_in, b_in, W_out, b_out)

    return out.reshape(B, T, DIM), idx


# speedup vs baseline: 1.2857x; 1.2857x over previous
import jax
import jax.numpy as jnp
from jax.experimental import pallas as pl

_CODE_DIMS = 13
_NB = 4


def _vq_kernel(x_ref, win_ref, bin_ref, wout_ref, bout_ref, out_ref, idx_ref):
    rows, dim = x_ref.shape
    t = idx_ref.shape[1]
    nb = rows // t
    x = x_ref[...]
    z = jax.lax.dot_general(
        x, win_ref[...], (((1,), (1,)), ((), ())),
        preferred_element_type=jnp.float32,
    )
    z = z + bin_ref[...][None, :]
    bits = (z > 0).astype(jnp.float32)
    q = bits * 2.0 - 1.0
    out = jnp.dot(q, wout_ref[...], preferred_element_type=jnp.float32)
    out_ref[...] = out + bout_ref[...][None, :]
    col = jax.lax.broadcasted_iota(jnp.int32, (1, _CODE_DIMS), 1)
    w_idx = jnp.exp2((_CODE_DIMS - 1 - col).astype(jnp.float32))
    i = pl.program_id(0)
    for r in range(nb):
        bits_r = bits[r * t:(r + 1) * t, :]
        idx_f = jax.lax.dot_general(
            w_idx, bits_r, (((1,), (1,)), ((), ())),
            preferred_element_type=jnp.float32,
        )
        idx_ref[pl.ds(i * nb + r, 1), :] = idx_f.astype(jnp.int32)


def kernel(x, W_in, b_in, W_out, b_out):
    B, T, DIM = x.shape
    rows = _NB * T
    out, idx = pl.pallas_call(
        _vq_kernel,
        grid=(B // _NB,),
        in_specs=[
            pl.BlockSpec((rows, DIM), lambda i: (i, 0)),
            pl.BlockSpec((_CODE_DIMS, DIM), lambda i: (0, 0)),
            pl.BlockSpec((_CODE_DIMS,), lambda i: (0,)),
            pl.BlockSpec((_CODE_DIMS, DIM), lambda i: (0, 0)),
            pl.BlockSpec((DIM,), lambda i: (0,)),
        ],
        out_specs=[
            pl.BlockSpec((rows, DIM), lambda i: (i, 0)),
            pl.BlockSpec((B, T), lambda i: (0, 0)),
        ],
        out_shape=[
            jax.ShapeDtypeStruct((B * T, DIM), jnp.float32),
            jax.ShapeDtypeStruct((B, T), jnp.int32),
        ],
    )(x.reshape(B * T, DIM), W_in.T, b_in, W_out, b_out)

    return out.reshape(B, T, DIM), idx
